# Initial kernel scaffold; baseline (speedup 1.0000x reference)
import jax
import jax.numpy as jnp
from jax.experimental import pallas as pl


def kernel(x, W):
    raise NotImplementedError("write your pallas kernel here")


# speedup vs baseline: 1.4762x; 1.4762x over previous
import functools

import jax
import jax.numpy as jnp
from jax import lax
from jax.experimental import pallas as pl
from jax.experimental.pallas import tpu as pltpu
from jax.experimental.pallas import tpu_sc as plsc

BATCH = 4096
HIST = 200
EMBED_DIM = 32

_B = BATCH * HIST
_NC = 2
_NS = 16
_NW = _NC * _NS
_PER_W = _B // _NW
_CH = 1600
_NCH = _PER_W // _CH

_mesh = plsc.VectorSubcoreMesh(core_axis_name="c", subcore_axis_name="s")


@functools.partial(
    pl.kernel,
    mesh=_mesh,
    out_type=jax.ShapeDtypeStruct((_B, EMBED_DIM), jnp.float32),
    scratch_types=[
        pltpu.VMEM((_CH,), jnp.int32),
        pltpu.VMEM((_CH, EMBED_DIM), jnp.float32),
        pltpu.SemaphoreType.DMA,
    ],
    compiler_params=pltpu.CompilerParams(use_tc_tiling_on_sc=False),
)
def _embed_sc(x_hbm, w_hbm, out_hbm, idx_v, rows_v, sem):
    wid = lax.axis_index("s") * _NC + lax.axis_index("c")
    base = pl.multiple_of(wid * _PER_W, _PER_W)

    def body(i, carry):
        off = pl.multiple_of(base + i * _CH, _CH)
        pltpu.sync_copy(x_hbm.at[pl.ds(off, _CH)], idx_v)
        pltpu.async_copy(w_hbm.at[idx_v], rows_v, sem).wait()
        pltpu.sync_copy(rows_v, out_hbm.at[pl.ds(off, _CH)])
        return carry

    lax.fori_loop(0, _NCH, body, 0)


def kernel(x, W):
    out = _embed_sc(x.reshape(_B), W)
    return out.reshape(BATCH, HIST, EMBED_DIM)


# speedup vs baseline: 1.4963x; 1.0136x over previous
import functools

import jax
import jax.numpy as jnp
from jax import lax
from jax.experimental import pallas as pl
from jax.experimental.pallas import tpu as pltpu
from jax.experimental.pallas import tpu_sc as plsc

BATCH = 4096
HIST = 200
EMBED_DIM = 32

_B = BATCH * HIST
_NC = 2
_NS = 16
_NW = _NC * _NS
_PER_W = _B // _NW
_NBUF = 4
_CH = 800
_NCH = _PER_W // _CH
_NGRP = _NCH // _NBUF

_mesh = plsc.VectorSubcoreMesh(core_axis_name="c", subcore_axis_name="s")


@functools.partial(
    pl.kernel,
    mesh=_mesh,
    out_type=jax.ShapeDtypeStruct((_B, EMBED_DIM), jnp.float32),
    scratch_types=[
        pltpu.VMEM((_PER_W,), jnp.int32),
        pltpu.VMEM((_NBUF, _CH, EMBED_DIM), jnp.float32),
        [pltpu.SemaphoreType.DMA] * _NBUF,
        [pltpu.SemaphoreType.DMA] * _NBUF,
    ],
    compiler_params=pltpu.CompilerParams(use_tc_tiling_on_sc=False),
)
def _embed_sc(x_hbm, w_hbm, out_hbm, idx_v, rows_v, gsems, ssems):
    wid = lax.axis_index("s") * _NC + lax.axis_index("c")
    base = pl.multiple_of(wid * _PER_W, _PER_W)

    pltpu.sync_copy(x_hbm.at[pl.ds(base, _PER_W)], idx_v)

    def group(j, carry):
        i0 = j * _NBUF
        gathers = []
        for b in range(_NBUF):
            goff = pl.multiple_of((i0 + b) * _CH, _CH)
            gathers.append(pltpu.async_copy(
                w_hbm.at[idx_v.at[pl.ds(goff, _CH)]], rows_v.at[b], gsems[b]))
        stores = []
        for b in range(_NBUF):
            gathers[b].wait()
            soff = pl.multiple_of(base + (i0 + b) * _CH, _CH)
            stores.append(pltpu.async_copy(
                rows_v.at[b], out_hbm.at[pl.ds(soff, _CH)], ssems[b]))
        for b in range(_NBUF):
            stores[b].wait()
        return carry

    lax.fori_loop(0, _NGRP, group, 0)


def kernel(x, W):
    out = _embed_sc(x.reshape(_B), W)
    return out.reshape(BATCH, HIST, EMBED_DIM)
